# baseline (speedup 1.0000x reference)
import functools

import jax
import jax.numpy as jnp
from jax import lax
from jax.experimental import pallas as pl
from jax.experimental.pallas import tpu as pltpu
from jax.experimental.pallas import tpu_sc as plsc

NUM_EXPERTS = 64
TOP_K = 8
BLK_M = 1024
NUM_CORES = 2
NUM_SUBCORES = 16
N_WORKERS = NUM_CORES * NUM_SUBCORES
LANES = 16


def _logits_body(x_ref, w_ref, b_ref, out_ref):
    out_ref[...] = jax.lax.dot_general(
        w_ref[...].astype(jnp.bfloat16), x_ref[...].astype(jnp.bfloat16),
        (((1,), (1,)), ((), ())),
        preferred_element_type=jnp.float32,
    ) + b_ref[...]


def _logits_tc(xr, W, b, n_chunk, d_model, blk_off):
    return pl.pallas_call(
        _logits_body,
        grid=(n_chunk // BLK_M,),
        in_specs=[
            pl.BlockSpec((BLK_M, d_model), lambda i: (i + blk_off, 0)),
            pl.BlockSpec((NUM_EXPERTS, d_model), lambda i: (0, 0)),
            pl.BlockSpec((NUM_EXPERTS, 1), lambda i: (0, 0)),
        ],
        out_specs=pl.BlockSpec((NUM_EXPERTS, BLK_M), lambda i: (0, i)),
        out_shape=jax.ShapeDtypeStruct((NUM_EXPERTS, n_chunk), jnp.float32),
        compiler_params=pltpu.CompilerParams(
            dimension_semantics=("arbitrary",),
        ),
    )(xr, W, b.reshape(NUM_EXPERTS, 1))


def _make_topk_sc(n_tokens):
    tok_per_w = n_tokens // N_WORKERS
    n_groups = tok_per_w // LANES
    mesh = plsc.VectorSubcoreMesh(core_axis_name="c", subcore_axis_name="s")

    @functools.partial(
        pl.kernel,
        mesh=mesh,
        out_type=[
            jax.ShapeDtypeStruct((n_tokens, TOP_K), jnp.float32),
            jax.ShapeDtypeStruct((n_tokens, TOP_K), jnp.int32),
        ],
        scratch_types=[
            pltpu.VMEM((NUM_EXPERTS, tok_per_w), jnp.float32),
            pltpu.VMEM((tok_per_w, TOP_K), jnp.float32),
            pltpu.VMEM((tok_per_w, TOP_K), jnp.int32),
        ],
        compiler_params=pltpu.CompilerParams(needs_layout_passes=False),
    )
    def topk_sc(lg_hbm, w_hbm, i_hbm, lg_v, wo_v, io_v):
        cid = lax.axis_index("c")
        sid = lax.axis_index("s")
        wid = sid * NUM_CORES + cid
        base = wid * tok_per_w
        pltpu.sync_copy(lg_hbm.at[:, pl.ds(base, tok_per_w)], lg_v)

        lane = lax.broadcasted_iota(jnp.int32, (LANES,), 0)
        neg_inf = jnp.full((LANES,), -jnp.inf, jnp.float32)

        def group(g, carry):
            start = g * LANES
            tok = start + lane
            vals = []
            idxs = []
            for _ in range(TOP_K):
                m = lg_v[0, pl.ds(start, LANES)]
                am = jnp.zeros((LANES,), jnp.int32)
                for e in range(1, NUM_EXPERTS):
                    v = lg_v[e, pl.ds(start, LANES)]
                    c = v > m
                    m = jnp.where(c, v, m)
                    am = jnp.where(c, e, am)
                vals.append(m)
                idxs.append(am)
                plsc.store_scatter(lg_v, [am, tok], neg_inf)
            es = [jnp.exp(v - vals[0]) for v in vals]
            s = es[0]
            for e_r in es[1:]:
                s = s + e_r
            for r in range(TOP_K):
                col = jnp.full((LANES,), r, jnp.int32)
                plsc.store_scatter(wo_v, [tok, col], es[r] / s)
                plsc.store_scatter(io_v, [tok, col], idxs[r])
            return carry

        lax.fori_loop(0, n_groups, group, 0)
        pltpu.sync_copy(wo_v, w_hbm.at[pl.ds(base, tok_per_w)])
        pltpu.sync_copy(io_v, i_hbm.at[pl.ds(base, tok_per_w)])

    return topk_sc


N_CHUNKS = 2


def kernel(x, W, b):
    B, T, d_model = x.shape
    n_tokens = B * T
    xr = x.reshape(n_tokens, d_model)
    chunk = n_tokens // N_CHUNKS
    topk_sc = _make_topk_sc(chunk)
    logits = [
        _logits_tc(xr, W, b, chunk, d_model, c * (chunk // BLK_M))
        for c in range(N_CHUNKS)
    ]
    outs = [topk_sc(lg) for lg in logits]
    weights = jnp.concatenate([o[0] for o in outs], axis=0)
    indices = jnp.concatenate([o[1] for o in outs], axis=0)
    aux_loss = jnp.array(0.0, dtype=jnp.float32)
    return (weights.reshape(B, T, TOP_K), indices.reshape(B, T, TOP_K),
            aux_loss)


# speedup vs baseline: 1.1839x; 1.1839x over previous
import functools

import jax
import jax.numpy as jnp
from jax import lax
from jax.experimental import pallas as pl
from jax.experimental.pallas import tpu as pltpu
from jax.experimental.pallas import tpu_sc as plsc

NUM_EXPERTS = 64
TOP_K = 8
BLK_M = 512
NUM_CORES = 2
NUM_SUBCORES = 16
N_WORKERS = NUM_CORES * NUM_SUBCORES
LANES = 16


def _fused_body(x_ref, w_ref, b_ref, wout_ref, iout_ref):
    run = jax.lax.dot_general(
        w_ref[...].astype(jnp.bfloat16), x_ref[...].astype(jnp.bfloat16),
        (((1,), (1,)), ((), ())),
        preferred_element_type=jnp.float32,
    ) + b_ref[...]
    m = run.shape[1]
    iota0 = jax.lax.broadcasted_iota(jnp.int32, (NUM_EXPERTS, m), 0)
    top_v = []
    top_i = []
    for _ in range(TOP_K):
        vt = run
        it = iota0
        n = NUM_EXPERTS
        while n > 1:
            h = n // 2
            va, vb = vt[:h], vt[h:]
            ia, ib = it[:h], it[h:]
            c = vb > va
            vt = jnp.where(c, vb, va)
            it = jnp.where(c, ib, ia)
            n = h
        top_v.append(vt)
        top_i.append(it)
        run = jnp.where(iota0 == it, -jnp.inf, run)
    vals = jnp.concatenate(top_v, axis=0)
    e = jnp.exp(vals - vals[:1])
    w = e / jnp.sum(e, axis=0, keepdims=True)
    wout_ref[...] = w
    iout_ref[...] = jnp.concatenate(top_i, axis=0)


def _fused_tc(xr, W, b, n_chunk, d_model, blk_off):
    return pl.pallas_call(
        _fused_body,
        grid=(n_chunk // BLK_M,),
        in_specs=[
            pl.BlockSpec((BLK_M, d_model), lambda i: (i + blk_off, 0)),
            pl.BlockSpec((NUM_EXPERTS, d_model), lambda i: (0, 0)),
            pl.BlockSpec((NUM_EXPERTS, 1), lambda i: (0, 0)),
        ],
        out_specs=[
            pl.BlockSpec((TOP_K, BLK_M), lambda i: (0, i)),
            pl.BlockSpec((TOP_K, BLK_M), lambda i: (0, i)),
        ],
        out_shape=[
            jax.ShapeDtypeStruct((TOP_K, n_chunk), jnp.float32),
            jax.ShapeDtypeStruct((TOP_K, n_chunk), jnp.int32),
        ],
        compiler_params=pltpu.CompilerParams(
            dimension_semantics=("arbitrary",),
        ),
    )(xr, W, b.reshape(NUM_EXPERTS, 1))


def _logits_body(x_ref, w_ref, b_ref, out_ref):
    out_ref[...] = jax.lax.dot_general(
        w_ref[...].astype(jnp.bfloat16), x_ref[...].astype(jnp.bfloat16),
        (((1,), (1,)), ((), ())),
        preferred_element_type=jnp.float32,
    ) + b_ref[...]


def _logits_tc(xr, W, b, n_chunk, d_model, blk_off):
    return pl.pallas_call(
        _logits_body,
        grid=(n_chunk // BLK_M,),
        in_specs=[
            pl.BlockSpec((BLK_M, d_model), lambda i: (i + blk_off, 0)),
            pl.BlockSpec((NUM_EXPERTS, d_model), lambda i: (0, 0)),
            pl.BlockSpec((NUM_EXPERTS, 1), lambda i: (0, 0)),
        ],
        out_specs=pl.BlockSpec((NUM_EXPERTS, BLK_M), lambda i: (0, i)),
        out_shape=jax.ShapeDtypeStruct((NUM_EXPERTS, n_chunk), jnp.float32),
        compiler_params=pltpu.CompilerParams(
            dimension_semantics=("arbitrary",),
        ),
    )(xr, W, b.reshape(NUM_EXPERTS, 1))


def _make_topk_sc(n_chunk):
    tok_per_w = n_chunk // N_WORKERS
    n_groups = tok_per_w // LANES
    mesh = plsc.VectorSubcoreMesh(core_axis_name="c", subcore_axis_name="s")

    @functools.partial(
        pl.kernel,
        mesh=mesh,
        out_type=[
            jax.ShapeDtypeStruct((n_chunk, TOP_K), jnp.float32),
            jax.ShapeDtypeStruct((n_chunk, TOP_K), jnp.int32),
        ],
        scratch_types=[
            pltpu.VMEM((NUM_EXPERTS, tok_per_w), jnp.float32),
            pltpu.VMEM((tok_per_w, TOP_K), jnp.float32),
            pltpu.VMEM((tok_per_w, TOP_K), jnp.int32),
        ],
        compiler_params=pltpu.CompilerParams(needs_layout_passes=False),
    )
    def topk_sc(lg_hbm, w_hbm, i_hbm, lg_v, wo_v, io_v):
        cid = lax.axis_index("c")
        sid = lax.axis_index("s")
        wid = sid * NUM_CORES + cid
        base = wid * tok_per_w
        pltpu.sync_copy(lg_hbm.at[:, pl.ds(base, tok_per_w)], lg_v)

        lane = lax.broadcasted_iota(jnp.int32, (LANES,), 0)
        neg_inf = jnp.full((LANES,), -jnp.inf, jnp.float32)
        ones = jnp.full((LANES,), 1, jnp.int32)
        zeros = jnp.zeros((LANES,), jnp.int32)

        def group(g, carry):
            start = g * LANES
            tok = start + lane
            vals = []
            idxs = []
            for _ in range(TOP_K):
                vt = []
                it = []
                for p in range(NUM_EXPERTS // 2):
                    va = lg_v[2 * p, pl.ds(start, LANES)]
                    vb = lg_v[2 * p + 1, pl.ds(start, LANES)]
                    c = vb > va
                    vt.append(jnp.where(c, vb, va))
                    it.append(2 * p + jnp.where(c, ones, zeros))
                while len(vt) > 1:
                    nv, ni = [], []
                    for p in range(len(vt) // 2):
                        va, vb = vt[2 * p], vt[2 * p + 1]
                        ia, ib = it[2 * p], it[2 * p + 1]
                        c = vb > va
                        nv.append(jnp.where(c, vb, va))
                        ni.append(jnp.where(c, ib, ia))
                    vt, it = nv, ni
                m, am = vt[0], it[0]
                vals.append(m)
                idxs.append(am)
                plsc.store_scatter(lg_v, [am, tok], neg_inf)
            es = [jnp.exp(v - vals[0]) for v in vals]
            s = es[0]
            for e_r in es[1:]:
                s = s + e_r
            for r in range(TOP_K):
                col = jnp.full((LANES,), r, jnp.int32)
                plsc.store_scatter(wo_v, [tok, col], es[r] / s)
                plsc.store_scatter(io_v, [tok, col], idxs[r])
            return carry

        lax.fori_loop(0, n_groups, group, 0)
        pltpu.sync_copy(wo_v, w_hbm.at[pl.ds(base, tok_per_w)])
        pltpu.sync_copy(io_v, i_hbm.at[pl.ds(base, tok_per_w)])

    return topk_sc


N_CHUNKS = 2


def kernel(x, W, b):
    B, T, d_model = x.shape
    n_tokens = B * T
    xr = x.reshape(n_tokens, d_model)
    chunk = n_tokens // N_CHUNKS
    lg_a = _logits_tc(xr, W, b, chunk, d_model, 0)
    out_a = _make_topk_sc(chunk)(lg_a)
    out_b = _fused_tc(xr, W, b, chunk, d_model, chunk // BLK_M)
    weights = jnp.concatenate([out_a[0], out_b[0].T], axis=0)
    indices = jnp.concatenate([out_a[1], out_b[1].T], axis=0)
    aux_loss = jnp.array(0.0, dtype=jnp.float32)
    return (weights.reshape(B, T, TOP_K), indices.reshape(B, T, TOP_K),
            aux_loss)


# speedup vs baseline: 1.2338x; 1.0421x over previous
import functools

import jax
import jax.numpy as jnp
from jax import lax
from jax.experimental import pallas as pl
from jax.experimental.pallas import tpu as pltpu
from jax.experimental.pallas import tpu_sc as plsc

NUM_EXPERTS = 64
TOP_K = 8
BLK_M = 512
NUM_CORES = 2
NUM_SUBCORES = 16
N_WORKERS = NUM_CORES * NUM_SUBCORES
LANES = 16


def _fused_body(x_ref, w_ref, b_ref, wout_ref, iout_ref):
    run = jax.lax.dot_general(
        w_ref[...].astype(jnp.bfloat16), x_ref[...].astype(jnp.bfloat16),
        (((1,), (1,)), ((), ())),
        preferred_element_type=jnp.float32,
    ) + b_ref[...]
    m = run.shape[1]
    iota0 = jax.lax.broadcasted_iota(jnp.int32, (NUM_EXPERTS, m), 0)
    top_v = []
    top_i = []
    for _ in range(TOP_K):
        vt = run
        it = iota0
        n = NUM_EXPERTS
        while n > 1:
            h = n // 2
            va, vb = vt[:h], vt[h:]
            ia, ib = it[:h], it[h:]
            c = vb > va
            vt = jnp.where(c, vb, va)
            it = jnp.where(c, ib, ia)
            n = h
        top_v.append(vt)
        top_i.append(it)
        run = jnp.where(iota0 == it, -jnp.inf, run)
    vals = jnp.concatenate(top_v, axis=0)
    e = jnp.exp(vals - vals[:1])
    w = e / jnp.sum(e, axis=0, keepdims=True)
    wout_ref[...] = w
    iout_ref[...] = jnp.concatenate(top_i, axis=0)


def _fused_tc(xr, W, b, n_chunk, d_model, blk_off):
    return pl.pallas_call(
        _fused_body,
        grid=(n_chunk // BLK_M,),
        in_specs=[
            pl.BlockSpec((BLK_M, d_model), lambda i: (i + blk_off, 0)),
            pl.BlockSpec((NUM_EXPERTS, d_model), lambda i: (0, 0)),
            pl.BlockSpec((NUM_EXPERTS, 1), lambda i: (0, 0)),
        ],
        out_specs=[
            pl.BlockSpec((TOP_K, BLK_M), lambda i: (0, i)),
            pl.BlockSpec((TOP_K, BLK_M), lambda i: (0, i)),
        ],
        out_shape=[
            jax.ShapeDtypeStruct((TOP_K, n_chunk), jnp.float32),
            jax.ShapeDtypeStruct((TOP_K, n_chunk), jnp.int32),
        ],
        compiler_params=pltpu.CompilerParams(
            dimension_semantics=("arbitrary",),
        ),
    )(xr, W, b.reshape(NUM_EXPERTS, 1))


def _logits_body(x_ref, w_ref, b_ref, out_ref):
    out_ref[...] = jax.lax.dot_general(
        w_ref[...].astype(jnp.bfloat16), x_ref[...].astype(jnp.bfloat16),
        (((1,), (1,)), ((), ())),
        preferred_element_type=jnp.float32,
    ) + b_ref[...]


def _logits_tc(xr, W, b, n_chunk, d_model, blk_off):
    return pl.pallas_call(
        _logits_body,
        grid=(n_chunk // BLK_M,),
        in_specs=[
            pl.BlockSpec((BLK_M, d_model), lambda i: (i + blk_off, 0)),
            pl.BlockSpec((NUM_EXPERTS, d_model), lambda i: (0, 0)),
            pl.BlockSpec((NUM_EXPERTS, 1), lambda i: (0, 0)),
        ],
        out_specs=pl.BlockSpec((NUM_EXPERTS, BLK_M), lambda i: (0, i)),
        out_shape=jax.ShapeDtypeStruct((NUM_EXPERTS, n_chunk), jnp.float32),
        compiler_params=pltpu.CompilerParams(
            dimension_semantics=("arbitrary",),
        ),
    )(xr, W, b.reshape(NUM_EXPERTS, 1))


def _make_topk_sc(n_chunk):
    tok_per_w = n_chunk // N_WORKERS
    n_groups = tok_per_w // LANES
    mesh = plsc.VectorSubcoreMesh(core_axis_name="c", subcore_axis_name="s")

    @functools.partial(
        pl.kernel,
        mesh=mesh,
        out_type=[
            jax.ShapeDtypeStruct((TOP_K, n_chunk), jnp.float32),
            jax.ShapeDtypeStruct((TOP_K, n_chunk), jnp.int32),
        ],
        scratch_types=[
            pltpu.VMEM((NUM_EXPERTS, tok_per_w), jnp.float32),
            pltpu.VMEM((TOP_K, tok_per_w), jnp.float32),
            pltpu.VMEM((TOP_K, tok_per_w), jnp.int32),
        ],
        compiler_params=pltpu.CompilerParams(needs_layout_passes=False),
    )
    def topk_sc(lg_hbm, w_hbm, i_hbm, lg_v, wo_v, io_v):
        cid = lax.axis_index("c")
        sid = lax.axis_index("s")
        wid = sid * NUM_CORES + cid
        base = wid * tok_per_w
        pltpu.sync_copy(lg_hbm.at[:, pl.ds(base, tok_per_w)], lg_v)

        lane = lax.broadcasted_iota(jnp.int32, (LANES,), 0)
        neg_inf = jnp.full((LANES,), -jnp.inf, jnp.float32)
        ones = jnp.full((LANES,), 1, jnp.int32)
        zeros = jnp.zeros((LANES,), jnp.int32)

        def group(g, carry):
            start = g * LANES
            tok = start + lane
            vals = []
            idxs = []
            for _ in range(TOP_K):
                vt = []
                it = []
                for p in range(NUM_EXPERTS // 2):
                    va = lg_v[2 * p, pl.ds(start, LANES)]
                    vb = lg_v[2 * p + 1, pl.ds(start, LANES)]
                    c = vb > va
                    vt.append(jnp.where(c, vb, va))
                    it.append(2 * p + jnp.where(c, ones, zeros))
                while len(vt) > 1:
                    nv, ni = [], []
                    for p in range(len(vt) // 2):
                        va, vb = vt[2 * p], vt[2 * p + 1]
                        ia, ib = it[2 * p], it[2 * p + 1]
                        c = vb > va
                        nv.append(jnp.where(c, vb, va))
                        ni.append(jnp.where(c, ib, ia))
                    vt, it = nv, ni
                m, am = vt[0], it[0]
                vals.append(m)
                idxs.append(am)
                plsc.store_scatter(lg_v, [am, tok], neg_inf)
            es = [jnp.exp(v - vals[0]) for v in vals]
            s = es[0]
            for e_r in es[1:]:
                s = s + e_r
            for r in range(TOP_K):
                wo_v[r, pl.ds(start, LANES)] = es[r] / s
                io_v[r, pl.ds(start, LANES)] = idxs[r]
            return carry

        lax.fori_loop(0, n_groups, group, 0)
        pltpu.sync_copy(wo_v, w_hbm.at[:, pl.ds(base, tok_per_w)])
        pltpu.sync_copy(io_v, i_hbm.at[:, pl.ds(base, tok_per_w)])

    return topk_sc


N_CHUNKS = 2


def kernel(x, W, b):
    B, T, d_model = x.shape
    n_tokens = B * T
    xr = x.reshape(n_tokens, d_model)
    chunk = n_tokens // N_CHUNKS
    lg_a = _logits_tc(xr, W, b, chunk, d_model, 0)
    out_a = _make_topk_sc(chunk)(lg_a)
    out_b = _fused_tc(xr, W, b, chunk, d_model, chunk // BLK_M)
    weights = jnp.concatenate([out_a[0], out_b[0]], axis=1).T
    indices = jnp.concatenate([out_a[1], out_b[1]], axis=1).T
    aux_loss = jnp.array(0.0, dtype=jnp.float32)
    return (weights.reshape(B, T, TOP_K), indices.reshape(B, T, TOP_K),
            aux_loss)
